# baseline (speedup 1.0000x reference)
import functools

import jax
import jax.numpy as jnp
from jax import lax
from jax.experimental import pallas as pl
from jax.experimental.pallas import tpu as pltpu
from jax.experimental.pallas import tpu_sc as plsc

_BATCH = 4096
_MAX_LEN = 200
_DIM = 50
_DPAD = 64
_NW = 32
_LT = _MAX_LEN // 8
_BT = _BATCH // 128
_PATCH = 1024


def _make_gather():
    mesh = plsc.VectorSubcoreMesh(core_axis_name="c", subcore_axis_name="s")

    @functools.partial(
        pl.kernel,
        out_type=jax.ShapeDtypeStruct((2, _DIM, _LT, _BT, 8, 128),
                                      jnp.float32),
        mesh=mesh,
        scratch_types=[
            pltpu.VMEM((8, 128), jnp.int32),
            pltpu.VMEM((_PATCH, _DPAD), jnp.float32),
            pltpu.VMEM((_DIM, 8, 128), jnp.float32),
            pltpu.SemaphoreType.DMA,
        ],
        compiler_params=pltpu.CompilerParams(
            use_tc_tiling_on_sc=False, needs_layout_passes=False
        ),
    )
    def gather(idx_t_hbm, idx_j_hbm, table_hbm, out_hbm, idx_v, win_v,
               tiles_v, sem):
        wid = lax.axis_index("s") * 2 + lax.axis_index("c")
        for h, idx_hbm in enumerate((idx_t_hbm, idx_j_hbm)):

            @pl.loop(0, _LT)
            def _patch(lt, idx_hbm=idx_hbm, h=h):
                pltpu.sync_copy(
                    idx_hbm.at[pl.ds(lt * 8, 8), pl.ds(wid * 128, 128)],
                    idx_v,
                )
                copies = [
                    pltpu.async_copy(
                        table_hbm.at[idx_v.at[li]],
                        win_v.at[pl.ds(li * 128, 128)],
                        sem,
                    )
                    for li in range(8)
                ]
                for cp in copies:
                    cp.wait()

                @pl.loop(0, _PATCH)
                def _row(r):
                    li = jnp.full((16,), r >> 7, jnp.int32)
                    bi = jnp.full((16,), r & 127, jnp.int32)
                    iota = lax.iota(jnp.int32, 16)
                    for d0 in (0, 16, 32, 34):
                        plsc.store_scatter(
                            tiles_v,
                            [iota + d0, li, bi],
                            win_v[r, pl.ds(d0, 16)],
                        )

                pltpu.sync_copy(tiles_v, out_hbm.at[h, :, lt, wid])

    return gather


_gather = _make_gather()


def kernel(idx_t, idx_j, emb_matrix):
    table_padded = jnp.pad(emb_matrix, ((0, 0), (0, _DPAD - _DIM)))
    out = _gather(idx_t.T, idx_j.T, table_padded)
    return out.transpose(0, 3, 5, 2, 4, 1).reshape(
        2, _BATCH, _MAX_LEN, _DIM
    )


# speedup vs baseline: 1.0359x; 1.0359x over previous
import functools

import jax
import jax.numpy as jnp
from jax import lax
from jax.experimental import pallas as pl
from jax.experimental.pallas import tpu as pltpu
from jax.experimental.pallas import tpu_sc as plsc

_BATCH = 4096
_MAX_LEN = 200
_DIM = 50
_DPAD = 64
_NW = 32
_LT = _MAX_LEN // 8
_HP = 512


def _make_gather():
    mesh = plsc.VectorSubcoreMesh(core_axis_name="c", subcore_axis_name="s")

    @functools.partial(
        pl.kernel,
        out_type=jax.ShapeDtypeStruct((2, _DIM, _LT, _NW, 8, 128),
                                      jnp.float32),
        mesh=mesh,
        scratch_types=[
            pltpu.VMEM((2, 4, 128), jnp.int32),
            pltpu.VMEM((2, _HP, _DPAD), jnp.float32),
            pltpu.VMEM((_DIM, 4, 128), jnp.float32),
            pltpu.SemaphoreType.DMA,
            pltpu.SemaphoreType.DMA,
        ],
        compiler_params=pltpu.CompilerParams(
            use_tc_tiling_on_sc=False, needs_layout_passes=False
        ),
    )
    def gather(idx_t_hbm, idx_j_hbm, table_hbm, out_hbm, idx_v, win_v,
               tiles_v, sem0, sem1):
        wid = lax.axis_index("s") * 2 + lax.axis_index("c")
        sems = (sem0, sem1)

        def stage(idx_hbm, irow, par):
            pltpu.sync_copy(
                idx_hbm.at[pl.ds(irow, 4), pl.ds(wid * 128, 128)],
                idx_v.at[par],
            )
            for li in range(4):
                pltpu.async_copy(
                    table_hbm.at[idx_v.at[par].at[li]],
                    win_v.at[par].at[pl.ds(li * 128, 128)],
                    sems[par],
                )

        def drain(par):
            for li in range(4):
                pltpu.make_async_copy(
                    table_hbm.at[idx_v.at[par].at[li]],
                    win_v.at[par].at[pl.ds(li * 128, 128)],
                    sems[par],
                ).wait()

        def build_and_store(h, lt, k, par):
            @pl.loop(0, _HP)
            def _row(r):
                li = jnp.full((16,), r >> 7, jnp.int32)
                bi = jnp.full((16,), r & 127, jnp.int32)
                iota = lax.iota(jnp.int32, 16)
                for d0 in (0, 16, 32, 34):
                    plsc.store_scatter(
                        tiles_v,
                        [iota + d0, li, bi],
                        win_v[par, r, pl.ds(d0, 16)],
                    )
            pltpu.sync_copy(
                tiles_v,
                out_hbm.at[h, :, lt, wid, pl.ds(k * 4, 4)],
            )

        for h, idx_hbm in enumerate((idx_t_hbm, idx_j_hbm)):
            stage(idx_hbm, 0, 0)

            @pl.loop(0, _LT)
            def _lt(lt, idx_hbm=idx_hbm, h=h):
                for k in (0, 1):
                    par = k
                    drain(par)
                    if k == 0:
                        stage(idx_hbm, lt * 8 + 4, 1 - par)
                    else:
                        @pl.when(lt + 1 < _LT)
                        def _():
                            stage(idx_hbm, (lt + 1) * 8, 1 - par)
                    build_and_store(h, lt, k, par)

    return gather


_gather = _make_gather()


def kernel(idx_t, idx_j, emb_matrix):
    table_padded = jnp.pad(emb_matrix, ((0, 0), (0, _DPAD - _DIM)))
    out = _gather(idx_t.T, idx_j.T, table_padded)
    return out.transpose(0, 3, 5, 2, 4, 1).reshape(
        2, _BATCH, _MAX_LEN, _DIM
    )


# speedup vs baseline: 1.0647x; 1.0278x over previous
import functools

import jax
import jax.numpy as jnp
from jax import lax
from jax.experimental import pallas as pl
from jax.experimental.pallas import tpu as pltpu
from jax.experimental.pallas import tpu_sc as plsc

_BATCH = 4096
_MAX_LEN = 200
_DIM = 50
_DPAD = 64
_NW = 32
_LT = _MAX_LEN // 8
_HP = 512


def _make_gather():
    mesh = plsc.VectorSubcoreMesh(core_axis_name="c", subcore_axis_name="s")

    @functools.partial(
        pl.kernel,
        out_type=jax.ShapeDtypeStruct((2, _DIM, _LT, _NW, 8, 128),
                                      jnp.float32),
        mesh=mesh,
        scratch_types=[
            pltpu.VMEM((2, 4, 128), jnp.int32),
            pltpu.VMEM((2, _HP, _DPAD), jnp.float32),
            pltpu.VMEM((2, _DIM, 4, 128), jnp.float32),
            pltpu.SemaphoreType.DMA,
            pltpu.SemaphoreType.DMA,
            pltpu.SemaphoreType.DMA,
            pltpu.SemaphoreType.DMA,
        ],
        compiler_params=pltpu.CompilerParams(
            use_tc_tiling_on_sc=False, needs_layout_passes=False
        ),
    )
    def gather(idx_t_hbm, idx_j_hbm, table_hbm, out_hbm, idx_v, win_v,
               tiles_v, sem0, sem1, osem0, osem1):
        wid = lax.axis_index("s") * 2 + lax.axis_index("c")
        sems = (sem0, sem1)
        osems = (osem0, osem1)

        def stage(idx_hbm, irow, par):
            pltpu.sync_copy(
                idx_hbm.at[pl.ds(irow, 4), pl.ds(wid * 128, 128)],
                idx_v.at[par],
            )
            for li in range(4):
                pltpu.async_copy(
                    table_hbm.at[idx_v.at[par].at[li]],
                    win_v.at[par].at[pl.ds(li * 128, 128)],
                    sems[par],
                )

        def drain_gathers(par):
            for li in range(4):
                pltpu.make_async_copy(
                    table_hbm.at[idx_v.at[par].at[li]],
                    win_v.at[par].at[pl.ds(li * 128, 128)],
                    sems[par],
                ).wait()

        def out_slice(h, lt, k):
            return out_hbm.at[h, :, lt, wid, pl.ds(k * 4, 4)]

        def drain_out(h, lt, k, par):
            pltpu.make_async_copy(
                tiles_v.at[par], out_slice(h, lt, k), osems[par]
            ).wait()

        def build_and_store(h, lt, k, par):
            @pl.loop(0, _HP)
            def _row(r):
                li = jnp.full((16,), r >> 7, jnp.int32)
                bi = jnp.full((16,), r & 127, jnp.int32)
                iota = lax.iota(jnp.int32, 16)
                for d0 in (0, 16, 32, 34):
                    plsc.store_scatter(
                        tiles_v.at[par],
                        [iota + d0, li, bi],
                        win_v[par, r, pl.ds(d0, 16)],
                    )
            pltpu.async_copy(tiles_v.at[par], out_slice(h, lt, k),
                             osems[par])

        for h, idx_hbm in enumerate((idx_t_hbm, idx_j_hbm)):
            stage(idx_hbm, 0, 0)

            @pl.loop(0, _LT)
            def _lt(lt, idx_hbm=idx_hbm, h=h):
                for k in (0, 1):
                    par = k
                    drain_gathers(par)
                    if k == 0:
                        stage(idx_hbm, lt * 8 + 4, 1 - par)
                    else:
                        @pl.when(lt + 1 < _LT)
                        def _():
                            stage(idx_hbm, (lt + 1) * 8, 1 - par)
                    if h == 0:
                        @pl.when(lt > 0)
                        def _():
                            drain_out(h, lt, k, par)
                    else:
                        drain_out(h, lt, k, par)
                    build_and_store(h, lt, k, par)

        for par in (0, 1):
            drain_out(1, _LT - 1, par, par)

    return gather


_gather = _make_gather()


def kernel(idx_t, idx_j, emb_matrix):
    table_padded = jnp.pad(emb_matrix, ((0, 0), (0, _DPAD - _DIM)))
    out = _gather(idx_t.T, idx_j.T, table_padded)
    return out.transpose(0, 3, 5, 2, 4, 1).reshape(
        2, _BATCH, _MAX_LEN, _DIM
    )


# speedup vs baseline: 1.5487x; 1.4546x over previous
import functools

import jax
import jax.numpy as jnp
from jax import lax
from jax.experimental import pallas as pl
from jax.experimental.pallas import tpu as pltpu
from jax.experimental.pallas import tpu_sc as plsc

_BATCH = 4096
_MAX_LEN = 200
_DIM = 50
_DPAD = 64
_NW = 32
_LT = _MAX_LEN // 8
_HP = 512


def _make_gather():
    mesh = plsc.VectorSubcoreMesh(core_axis_name="c", subcore_axis_name="s")

    @functools.partial(
        pl.kernel,
        out_type=jax.ShapeDtypeStruct((2, _DIM, _LT, _NW, 8, 128),
                                      jnp.float32),
        mesh=mesh,
        scratch_types=[
            pltpu.VMEM((2, 4, 128), jnp.int32),
            pltpu.VMEM((2, _HP, _DPAD), jnp.float32),
            pltpu.VMEM((2, _DIM, 4, 128), jnp.float32),
            pltpu.SemaphoreType.DMA,
            pltpu.SemaphoreType.DMA,
            pltpu.SemaphoreType.DMA,
            pltpu.SemaphoreType.DMA,
        ],
        compiler_params=pltpu.CompilerParams(
            use_tc_tiling_on_sc=False, needs_layout_passes=False
        ),
    )
    def gather(idx_t_hbm, idx_j_hbm, table_hbm, out_hbm, idx_v, win_v,
               tiles_v, sem0, sem1, osem0, osem1):
        wid = lax.axis_index("s") * 2 + lax.axis_index("c")
        sems = (sem0, sem1)
        osems = (osem0, osem1)

        def stage(idx_hbm, irow, par):
            pltpu.sync_copy(
                idx_hbm.at[pl.ds(irow, 4), pl.ds(wid * 128, 128)],
                idx_v.at[par],
            )
            for li in range(4):
                pltpu.async_copy(
                    table_hbm.at[idx_v.at[par].at[li]],
                    win_v.at[par].at[pl.ds(li * 128, 128)],
                    sems[par],
                )

        def drain_gathers(par):
            for li in range(4):
                pltpu.make_async_copy(
                    table_hbm.at[idx_v.at[par].at[li]],
                    win_v.at[par].at[pl.ds(li * 128, 128)],
                    sems[par],
                ).wait()

        def out_slice(h, lt, k):
            return out_hbm.at[h, :, lt, wid, pl.ds(k * 4, 4)]

        def drain_out(h, lt, k, par):
            pltpu.make_async_copy(
                tiles_v.at[par], out_slice(h, lt, k), osems[par]
            ).wait()

        def build_and_store(h, lt, k, par):
            win2d = win_v.at[par]
            tiles = tiles_v.at[par]

            @pl.loop(0, _HP // 4)
            def _rg(rg):
                r0 = rg * 4
                iota = lax.iota(jnp.int32, 16)
                ri = iota & 3
                di = iota >> 2
                row_vec = ri + r0
                li_vec = jnp.full((16,), r0 >> 7, jnp.int32)
                bi_vec = ri + (r0 & 127)
                for d0 in tuple(range(0, 48, 4)) + (46,):
                    d_vec = di + d0
                    vals = plsc.load_gather(win2d, [row_vec, d_vec])
                    plsc.store_scatter(tiles, [d_vec, li_vec, bi_vec], vals)

            pltpu.async_copy(tiles_v.at[par], out_slice(h, lt, k),
                             osems[par])

        for h, idx_hbm in enumerate((idx_t_hbm, idx_j_hbm)):
            stage(idx_hbm, 0, 0)

            @pl.loop(0, _LT)
            def _lt(lt, idx_hbm=idx_hbm, h=h):
                for k in (0, 1):
                    par = k
                    drain_gathers(par)
                    if k == 0:
                        stage(idx_hbm, lt * 8 + 4, 1 - par)
                    else:
                        @pl.when(lt + 1 < _LT)
                        def _():
                            stage(idx_hbm, (lt + 1) * 8, 1 - par)
                    if h == 0:
                        @pl.when(lt > 0)
                        def _():
                            drain_out(h, lt, k, par)
                    else:
                        drain_out(h, lt, k, par)
                    build_and_store(h, lt, k, par)

        for par in (0, 1):
            drain_out(1, _LT - 1, par, par)

    return gather


_gather = _make_gather()


def kernel(idx_t, idx_j, emb_matrix):
    table_padded = jnp.pad(emb_matrix, ((0, 0), (0, _DPAD - _DIM)))
    out = _gather(idx_t.T, idx_j.T, table_padded)
    return out.transpose(0, 3, 5, 2, 4, 1).reshape(
        2, _BATCH, _MAX_LEN, _DIM
    )


# speedup vs baseline: 1.5510x; 1.0015x over previous
import functools

import jax
import jax.numpy as jnp
from jax import lax
from jax.experimental import pallas as pl
from jax.experimental.pallas import tpu as pltpu
from jax.experimental.pallas import tpu_sc as plsc

_BATCH = 4096
_MAX_LEN = 200
_DIM = 50
_DPAD = 64
_NW = 32
_LT = _MAX_LEN // 8
_HP = 512


def _make_gather():
    mesh = plsc.VectorSubcoreMesh(core_axis_name="c", subcore_axis_name="s")

    @functools.partial(
        pl.kernel,
        out_type=jax.ShapeDtypeStruct((2, _DIM, _LT, _NW, 8, 128),
                                      jnp.float32),
        mesh=mesh,
        scratch_types=[
            pltpu.VMEM((2, 4, 128), jnp.int32),
            pltpu.VMEM((2, _HP, _DPAD), jnp.float32),
            pltpu.VMEM((2, _DIM, 4, 128), jnp.float32),
            pltpu.SemaphoreType.DMA,
            pltpu.SemaphoreType.DMA,
            pltpu.SemaphoreType.DMA,
            pltpu.SemaphoreType.DMA,
        ],
        compiler_params=pltpu.CompilerParams(
            use_tc_tiling_on_sc=False, needs_layout_passes=False
        ),
    )
    def gather(idx_t_hbm, idx_j_hbm, table_hbm, out_hbm, idx_v, win_v,
               tiles_v, sem0, sem1, osem0, osem1):
        wid = lax.axis_index("s") * 2 + lax.axis_index("c")
        sems = (sem0, sem1)
        osems = (osem0, osem1)

        def stage(idx_hbm, irow, par):
            pltpu.sync_copy(
                idx_hbm.at[pl.ds(irow, 4), pl.ds(wid * 128, 128)],
                idx_v.at[par],
            )
            for li in range(4):
                pltpu.async_copy(
                    table_hbm.at[idx_v.at[par].at[li]],
                    win_v.at[par].at[pl.ds(li * 128, 128)],
                    sems[par],
                )

        def drain_gathers(par):
            for li in range(4):
                pltpu.make_async_copy(
                    table_hbm.at[idx_v.at[par].at[li]],
                    win_v.at[par].at[pl.ds(li * 128, 128)],
                    sems[par],
                ).wait()

        def out_slice(h, lt, k):
            return out_hbm.at[h, :, lt, wid, pl.ds(k * 4, 4)]

        def drain_out(h, lt, k, par):
            pltpu.make_async_copy(
                tiles_v.at[par], out_slice(h, lt, k), osems[par]
            ).wait()

        def build_and_store(h, lt, k, par):
            win2d = win_v.at[par]
            tiles = tiles_v.at[par]

            @pl.loop(0, _HP // 4, unroll=4)
            def _rg(rg):
                r0 = rg * 4
                iota = lax.iota(jnp.int32, 16)
                ri = iota & 3
                di = iota >> 2
                row_vec = ri + r0
                li_vec = jnp.full((16,), r0 >> 7, jnp.int32)
                bi_vec = ri + (r0 & 127)
                for d0 in tuple(range(0, 48, 4)) + (46,):
                    d_vec = di + d0
                    vals = plsc.load_gather(win2d, [row_vec, d_vec])
                    plsc.store_scatter(tiles, [d_vec, li_vec, bi_vec], vals)

            pltpu.async_copy(tiles_v.at[par], out_slice(h, lt, k),
                             osems[par])

        for h, idx_hbm in enumerate((idx_t_hbm, idx_j_hbm)):
            stage(idx_hbm, 0, 0)

            @pl.loop(0, _LT)
            def _lt(lt, idx_hbm=idx_hbm, h=h):
                for k in (0, 1):
                    par = k
                    drain_gathers(par)
                    if k == 0:
                        stage(idx_hbm, lt * 8 + 4, 1 - par)
                    else:
                        @pl.when(lt + 1 < _LT)
                        def _():
                            stage(idx_hbm, (lt + 1) * 8, 1 - par)
                    if h == 0:
                        @pl.when(lt > 0)
                        def _():
                            drain_out(h, lt, k, par)
                    else:
                        drain_out(h, lt, k, par)
                    build_and_store(h, lt, k, par)

        for par in (0, 1):
            drain_out(1, _LT - 1, par, par)

    return gather


_gather = _make_gather()


def kernel(idx_t, idx_j, emb_matrix):
    table_padded = jnp.pad(emb_matrix, ((0, 0), (0, _DPAD - _DIM)))
    out = _gather(idx_t.T, idx_j.T, table_padded)
    return out.transpose(0, 3, 5, 2, 4, 1).reshape(
        2, _BATCH, _MAX_LEN, _DIM
    )


# speedup vs baseline: 1.6791x; 1.0826x over previous
import functools

import jax
import jax.numpy as jnp
from jax import lax
from jax.experimental import pallas as pl
from jax.experimental.pallas import tpu as pltpu
from jax.experimental.pallas import tpu_sc as plsc

_BATCH = 4096
_MAX_LEN = 200
_DIM = 50
_NW = 32
_LT = _MAX_LEN // 8
_HP = 512
_NG = 4 * _HP // 128
_VOCAB = 1000000
_GRAN = _VOCAB * _DIM // 16


def _make_gather():
    mesh = plsc.VectorSubcoreMesh(core_axis_name="c", subcore_axis_name="s")

    @functools.partial(
        pl.kernel,
        out_type=jax.ShapeDtypeStruct((2, _DIM, _LT, _NW, 8, 128),
                                      jnp.float32),
        mesh=mesh,
        scratch_types=[
            pltpu.VMEM((2, 4, 128), jnp.int32),
            pltpu.VMEM((2, _NG, 128), jnp.int32),
            pltpu.VMEM((2, _HP), jnp.int32),
            pltpu.VMEM((2, 4 * _HP, 16), jnp.float32),
            pltpu.VMEM((2, _DIM, 4, 128), jnp.float32),
            pltpu.SemaphoreType.DMA,
            pltpu.SemaphoreType.DMA,
            pltpu.SemaphoreType.DMA,
            pltpu.SemaphoreType.DMA,
        ],
        compiler_params=pltpu.CompilerParams(
            use_tc_tiling_on_sc=False, needs_layout_passes=False
        ),
    )
    def gather(idx_t_hbm, idx_j_hbm, table_hbm, out_hbm, idx_v, gidx_v,
               ph_v, win_v, tiles_v, sem0, sem1, osem0, osem1):
        wid = lax.axis_index("s") * 2 + lax.axis_index("c")
        sems = (sem0, sem1)
        osems = (osem0, osem1)

        def stage(idx_hbm, irow, par):
            pltpu.sync_copy(
                idx_hbm.at[pl.ds(irow, 4), pl.ds(wid * 128, 128)],
                idx_v.at[par],
            )
            iota = lax.iota(jnp.int32, 16)
            for li in range(4):
                for s in range(8):
                    r0 = li * 128 + s * 16
                    i = idx_v[par, li, pl.ds(s * 16, 16)]
                    g = (i * 25) >> 3
                    ph_v[par, pl.ds(r0, 16)] = (i * 2) & 15
                    pos = iota * 4 + (r0 * 4)
                    for j in range(4):
                        plsc.store_scatter(
                            gidx_v.at[par],
                            [(pos + j) >> 7, (pos + j) & 127],
                            g + j,
                        )
            for q in range(_NG):
                pltpu.async_copy(
                    table_hbm.at[gidx_v.at[par].at[q]],
                    win_v.at[par].at[pl.ds(q * 128, 128)],
                    sems[par],
                )

        def drain_gathers(par):
            for q in range(_NG):
                pltpu.make_async_copy(
                    table_hbm.at[gidx_v.at[par].at[q]],
                    win_v.at[par].at[pl.ds(q * 128, 128)],
                    sems[par],
                ).wait()

        def out_slice(h, lt, k):
            return out_hbm.at[h, :, lt, wid, pl.ds(k * 4, 4)]

        def drain_out(h, lt, k, par):
            pltpu.make_async_copy(
                tiles_v.at[par], out_slice(h, lt, k), osems[par]
            ).wait()

        def build_and_store(h, lt, k, par):
            win16 = win_v.at[par]
            tiles = tiles_v.at[par]
            phases = ph_v.at[par]

            @pl.loop(0, _HP // 4, unroll=4)
            def _rg(rg):
                r0 = rg * 4
                iota = lax.iota(jnp.int32, 16)
                ri = iota & 3
                di = iota >> 2
                row_vec = ri + r0
                ph_vec = plsc.load_gather(phases, [row_vec])
                wbase = ph_vec + (row_vec << 6) + di
                li_vec = jnp.full((16,), r0 >> 7, jnp.int32)
                bi_vec = ri + (r0 & 127)
                for d0 in tuple(range(0, 48, 4)) + (46,):
                    word = wbase + d0
                    vals = plsc.load_gather(
                        win16, [word >> 4, word & 15]
                    )
                    plsc.store_scatter(
                        tiles, [di + d0, li_vec, bi_vec], vals
                    )

            pltpu.async_copy(tiles_v.at[par], out_slice(h, lt, k),
                             osems[par])

        for h, idx_hbm in enumerate((idx_t_hbm, idx_j_hbm)):
            stage(idx_hbm, 0, 0)

            @pl.loop(0, _LT)
            def _lt(lt, idx_hbm=idx_hbm, h=h):
                for k in (0, 1):
                    par = k
                    drain_gathers(par)
                    if k == 0:
                        stage(idx_hbm, lt * 8 + 4, 1 - par)
                    else:
                        @pl.when(lt + 1 < _LT)
                        def _():
                            stage(idx_hbm, (lt + 1) * 8, 1 - par)
                    if h == 0:
                        @pl.when(lt > 0)
                        def _():
                            drain_out(h, lt, k, par)
                    else:
                        drain_out(h, lt, k, par)
                    build_and_store(h, lt, k, par)

        for par in (0, 1):
            drain_out(1, _LT - 1, par, par)

    return gather


_gather = _make_gather()


def kernel(idx_t, idx_j, emb_matrix):
    table_granules = emb_matrix.reshape(_GRAN, 16)
    out = _gather(idx_t.T, idx_j.T, table_granules)
    return out.transpose(0, 3, 5, 2, 4, 1).reshape(
        2, _BATCH, _MAX_LEN, _DIM
    )
